# Initial kernel scaffold; baseline (speedup 1.0000x reference)
import jax
import jax.numpy as jnp
from jax.experimental import pallas as pl


def kernel(inputs, embedding_variable):
    raise NotImplementedError("write your pallas kernel here")


# speedup vs baseline: 2.9661x; 2.9661x over previous
import functools

import jax
import jax.numpy as jnp
from jax import lax
from jax.experimental import pallas as pl
from jax.experimental.pallas import tpu as pltpu
from jax.experimental.pallas import tpu_sc as plsc

_EMBED = 128
_NC, _NS = 2, 16
_NW = _NC * _NS
_G = 128


def _sc_gather(idx3, table):
    nw, ng, g = idx3.shape
    b = nw * ng * g
    mesh = plsc.VectorSubcoreMesh(
        core_axis_name="c", subcore_axis_name="s",
        num_cores=_NC, num_subcores=_NS)

    @functools.partial(
        pl.kernel,
        out_type=jax.ShapeDtypeStruct((b, _EMBED), jnp.float32),
        mesh=mesh,
        scratch_types=[
            pltpu.VMEM((ng, g), jnp.int32),
            pltpu.VMEM((g, _EMBED), jnp.float32),
            pltpu.SemaphoreType.DMA,
        ],
    )
    def k(idx_hbm, table_hbm, out_hbm, idx_v, rows_v, sem):
        wid = lax.axis_index("s") * _NC + lax.axis_index("c")
        base = wid * (ng * g)
        pltpu.sync_copy(idx_hbm.at[wid], idx_v)

        def body(j, carry):
            pltpu.async_copy(table_hbm.at[idx_v.at[j]], rows_v, sem).wait()
            pltpu.sync_copy(rows_v, out_hbm.at[pl.ds(base + j * g, g)])
            return carry

        lax.fori_loop(0, ng, body, 0)

    return k(idx3, table)


def kernel(inputs, embedding_variable):
    batch, hist = inputs.shape
    idx3 = inputs.reshape(_NW, -1, _G).astype(jnp.int32)
    out = _sc_gather(idx3, embedding_variable)
    return out.reshape(batch, hist, _EMBED)


# speedup vs baseline: 3.3360x; 1.1247x over previous
import functools

import jax
import jax.numpy as jnp
from jax import lax
from jax.experimental import pallas as pl
from jax.experimental.pallas import tpu as pltpu
from jax.experimental.pallas import tpu_sc as plsc

_EMBED = 128
_NC, _NS = 2, 16
_NW = _NC * _NS
_G = 128
_NB = 5


def _sc_gather(idx3, table):
    nw, ng, g = idx3.shape
    b = nw * ng * g
    assert ng % _NB == 0
    mesh = plsc.VectorSubcoreMesh(
        core_axis_name="c", subcore_axis_name="s",
        num_cores=_NC, num_subcores=_NS)

    @functools.partial(
        pl.kernel,
        out_type=jax.ShapeDtypeStruct((b, _EMBED), jnp.float32),
        mesh=mesh,
        scratch_types=[
            pltpu.VMEM((ng, g), jnp.int32),
            [pltpu.VMEM((g, _EMBED), jnp.float32) for _ in range(_NB)],
            [pltpu.SemaphoreType.DMA for _ in range(_NB)],
            [pltpu.SemaphoreType.DMA for _ in range(_NB)],
        ],
    )
    def k(idx_hbm, table_hbm, out_hbm, idx_v, bufs, gsems, ssems):
        wid = lax.axis_index("s") * _NC + lax.axis_index("c")
        base = wid * (ng * g)
        pltpu.sync_copy(idx_hbm.at[wid], idx_v)

        _D = 3
        for i in range(_D):
            pltpu.async_copy(table_hbm.at[idx_v.at[i]], bufs[i], gsems[i])

        @pl.loop(0, ng, step=_NB)
        def _outer(t0):
            for i in range(_NB):
                t = t0 + i
                pltpu.make_async_copy(
                    table_hbm.at[idx_v.at[0]], bufs[i], gsems[i]).wait()
                pltpu.async_copy(
                    bufs[i], out_hbm.at[pl.ds(base + t * g, g)], ssems[i])
                tn = t + _D
                bn = (i + _D) % _NB

                @pl.when(tn < ng)
                def _():
                    @pl.when(tn >= _NB)
                    def _():
                        pltpu.make_async_copy(
                            bufs[bn], out_hbm.at[pl.ds(base, g)],
                            ssems[bn]).wait()

                    pltpu.async_copy(
                        table_hbm.at[idx_v.at[tn]], bufs[bn], gsems[bn])

        for i in range(_NB):
            pltpu.make_async_copy(
                bufs[i], out_hbm.at[pl.ds(base, g)], ssems[i]).wait()

    return k(idx3, table)


def kernel(inputs, embedding_variable):
    batch, hist = inputs.shape
    idx3 = inputs.reshape(_NW, -1, _G).astype(jnp.int32)
    out = _sc_gather(idx3, embedding_variable)
    return out.reshape(batch, hist, _EMBED)


# speedup vs baseline: 5.9409x; 1.7809x over previous
import functools

import jax
import jax.numpy as jnp
from jax import lax
from jax.experimental import pallas as pl
from jax.experimental.pallas import tpu as pltpu
from jax.experimental.pallas import tpu_sc as plsc

_EMBED = 128
_NC, _NS = 2, 16
_NW = _NC * _NS
_E = 4
_NB = 4
_D = 2


def _sc_gather(idx, table):
    batch, hist = idx.shape
    epw = batch // _NW
    nsup = epw // _E
    assert epw * _NW == batch and nsup * _E == epw and nsup % _NB == 0
    mesh = plsc.VectorSubcoreMesh(
        core_axis_name="c", subcore_axis_name="s",
        num_cores=_NC, num_subcores=_NS)

    @functools.partial(
        pl.kernel,
        out_type=jax.ShapeDtypeStruct((batch, hist, _EMBED), jnp.float32),
        mesh=mesh,
        compiler_params=pltpu.CompilerParams(use_tc_tiling_on_sc=True),
        scratch_types=[
            pltpu.VMEM((epw, hist), jnp.int32),
            [pltpu.VMEM((_E, hist, _EMBED), jnp.float32) for _ in range(_NB)],
            [pltpu.SemaphoreType.DMA for _ in range(_NB)],
            [pltpu.SemaphoreType.DMA for _ in range(_NB)],
        ],
    )
    def k(idx_hbm, table_hbm, out_hbm, idx_v, bufs, gsems, ssems):
        wid = lax.axis_index("s") * _NC + lax.axis_index("c")
        eb = wid * epw
        pltpu.sync_copy(idx_hbm.at[pl.ds(eb, epw)], idx_v)

        def fire(sup, slot):
            for e in range(_E):
                pltpu.async_copy(
                    table_hbm.at[idx_v.at[sup * _E + e]],
                    bufs[slot].at[e], gsems[slot])

        for i in range(_D):
            fire(i, i)

        @pl.loop(0, nsup, step=_NB)
        def _outer(s0):
            for i in range(_NB):
                sup = s0 + i
                pltpu.make_async_copy(
                    out_hbm.at[pl.ds(eb, _E)], bufs[i], gsems[i]).wait()
                pltpu.async_copy(
                    bufs[i], out_hbm.at[pl.ds(eb + sup * _E, _E)], ssems[i])
                sn = sup + _D
                bn = (i + _D) % _NB

                @pl.when(sn < nsup)
                def _():
                    @pl.when(sn >= _NB)
                    def _():
                        pltpu.make_async_copy(
                            bufs[bn], out_hbm.at[pl.ds(eb, _E)],
                            ssems[bn]).wait()

                    fire(sn, bn)

        for i in range(_NB):
            pltpu.make_async_copy(
                bufs[i], out_hbm.at[pl.ds(eb, _E)], ssems[i]).wait()

    return k(idx, table)


def kernel(inputs, embedding_variable):
    return _sc_gather(inputs.astype(jnp.int32), embedding_variable)


# speedup vs baseline: 10.7859x; 1.8155x over previous
import functools

import jax
import jax.numpy as jnp
from jax import lax
from jax.experimental import pallas as pl
from jax.experimental.pallas import tpu as pltpu
from jax.experimental.pallas import tpu_sc as plsc

_EMBED = 128
_NC, _NS = 2, 16
_NW = _NC * _NS
_G = 128
_NB = 5
_D = 3


def _sc_gather(idx_t, table):
    hist, batch = idx_t.shape
    nb = batch * hist
    assert batch % (_NW * _G) == 0 and hist % _NB == 0
    mesh = plsc.VectorSubcoreMesh(
        core_axis_name="c", subcore_axis_name="s",
        num_cores=_NC, num_subcores=_NS)

    @functools.partial(
        pl.kernel,
        out_type=jax.ShapeDtypeStruct((hist, batch, _EMBED), jnp.float32),
        mesh=mesh,
        compiler_params=pltpu.CompilerParams(use_tc_tiling_on_sc=True),
        scratch_types=[
            pltpu.VMEM((hist, _G), jnp.int32),
            [pltpu.VMEM((_G, _EMBED), jnp.float32) for _ in range(_NB)],
            [pltpu.SemaphoreType.DMA for _ in range(_NB)],
            [pltpu.SemaphoreType.DMA for _ in range(_NB)],
        ],
    )
    def k(idx_hbm, table_hbm, out3_hbm, idx_v, bufs, gsems, ssems):
        wid = lax.axis_index("s") * _NC + lax.axis_index("c")
        nbase = wid * _G
        out_hbm = out3_hbm.reshape(nb, _EMBED)
        pltpu.sync_copy(idx_hbm.at[:, pl.ds(nbase, _G)], idx_v)

        for i in range(_D):
            pltpu.async_copy(table_hbm.at[idx_v.at[i]], bufs[i], gsems[i])

        @pl.loop(0, hist, step=_NB)
        def _outer(t0):
            for i in range(_NB):
                h = t0 + i
                pltpu.make_async_copy(
                    table_hbm.at[idx_v.at[0]], bufs[i], gsems[i]).wait()
                pltpu.async_copy(
                    bufs[i], out_hbm.at[pl.ds(h * batch + nbase, _G)],
                    ssems[i])
                hn = h + _D
                bn = (i + _D) % _NB

                @pl.when(hn < hist)
                def _():
                    @pl.when(hn >= _NB)
                    def _():
                        pltpu.make_async_copy(
                            bufs[bn], out_hbm.at[pl.ds(nbase, _G)],
                            ssems[bn]).wait()

                    pltpu.async_copy(
                        table_hbm.at[idx_v.at[hn]], bufs[bn], gsems[bn])

        for i in range(_NB):
            pltpu.make_async_copy(
                bufs[i], out_hbm.at[pl.ds(nbase, _G)], ssems[i]).wait()

    return k(idx_t, table)


def kernel(inputs, embedding_variable):
    idx_t = inputs.T.astype(jnp.int32)
    out_t = _sc_gather(idx_t, embedding_variable)
    return out_t.transpose(1, 0, 2)
